# baseline (speedup 1.0000x reference)
import functools

import jax
import jax.numpy as jnp
from jax import lax
from jax.experimental import pallas as pl
from jax.experimental.pallas import tpu as pltpu
from jax.experimental.pallas import tpu_sc as plsc

_NC = 2
_NS = 16
_L = 16
_CHUNK = 128


@functools.cache
def _build(B: int, D: int):
    NW = _NC * _NS
    assert B % (NW * _L) == 0
    b_per_w = B // NW
    n_chunks = b_per_w // _CHUNK
    n_groups = b_per_w // _L
    mesh = plsc.VectorSubcoreMesh(core_axis_name="c", subcore_axis_name="s")

    @functools.partial(
        pl.kernel,
        out_type=jax.ShapeDtypeStruct((B,), jnp.float32),
        mesh=mesh,
        compiler_params=pltpu.CompilerParams(use_tc_tiling_on_sc=False,
                                             needs_layout_passes=False),
        scratch_types=[
            pltpu.VMEM((n_chunks, _CHUNK), jnp.int32),
            pltpu.VMEM((n_chunks, _CHUNK), jnp.int32),
            pltpu.VMEM((b_per_w, D), jnp.float32),
            pltpu.VMEM((b_per_w, D), jnp.float32),
            pltpu.VMEM((b_per_w,), jnp.float32),
            pltpu.SemaphoreType.DMA,
        ],
    )
    def k(uidx_hbm, iidx_hbm, utab_hbm, itab_hbm, out_hbm,
          uidx_v, iidx_v, urows_v, irows_v, out_v, sem):
        wid = lax.axis_index("s") * _NC + lax.axis_index("c")
        base = wid * b_per_w

        for j in range(n_chunks):
            pltpu.sync_copy(uidx_hbm.at[pl.ds(base + j * _CHUNK, _CHUNK)],
                            uidx_v.at[j])
            pltpu.sync_copy(iidx_hbm.at[pl.ds(base + j * _CHUNK, _CHUNK)],
                            iidx_v.at[j])
        copies = []
        for j in range(n_chunks):
            copies.append(pltpu.async_copy(
                utab_hbm.at[uidx_v.at[j]],
                urows_v.at[pl.ds(j * _CHUNK, _CHUNK)], sem))
            copies.append(pltpu.async_copy(
                itab_hbm.at[iidx_v.at[j]],
                irows_v.at[pl.ds(j * _CHUNK, _CHUNK)], sem))
        for c in copies:
            c.wait()

        lanes = lax.iota(jnp.int32, _L)
        cols = [jnp.full((_L,), d, jnp.int32) for d in range(D)]

        def body(g, carry):
            row = jnp.full((_L,), g * _L, jnp.int32) + lanes
            accs = [jnp.zeros((_L,), jnp.float32) for _ in range(4)]
            for d in range(D):
                u = plsc.load_gather(urows_v, [row, cols[d]])
                v = plsc.load_gather(irows_v, [row, cols[d]])
                accs[d % 4] = accs[d % 4] + u * v
            out_v[pl.ds(g * _L, _L)] = (accs[0] + accs[1]) + (accs[2] + accs[3])
            return carry

        lax.fori_loop(0, n_groups, body, 0)
        pltpu.sync_copy(out_v, out_hbm.at[pl.ds(base, b_per_w)])

    return k


def kernel(user_item_tuple, user_table, item_table):
    uid = user_item_tuple[:, 0].astype(jnp.int32)
    iid = user_item_tuple[:, 1].astype(jnp.int32)
    return _build(uid.shape[0], user_table.shape[1])(
        uid, iid, user_table, item_table)


# speedup vs baseline: 1.0025x; 1.0025x over previous
import functools

import jax
import jax.numpy as jnp
from jax import lax
from jax.experimental import pallas as pl
from jax.experimental.pallas import tpu as pltpu
from jax.experimental.pallas import tpu_sc as plsc

_NC = 2
_NS = 16
_L = 16
_CHUNK = 128
_LANE = 128


@functools.cache
def _build(B: int, D: int):
    NW = _NC * _NS
    assert B % (NW * _CHUNK) == 0 and _LANE % D == 0
    b_per_w = B // NW
    n_chunks = b_per_w // _CHUNK
    pack = _LANE // D
    shift = pack.bit_length() - 1
    assert 1 << shift == pack
    groups = _CHUNK // _L
    mesh = plsc.VectorSubcoreMesh(core_axis_name="c", subcore_axis_name="s")

    @functools.partial(
        pl.kernel,
        out_type=jax.ShapeDtypeStruct((B,), jnp.float32),
        mesh=mesh,
        compiler_params=pltpu.CompilerParams(use_tc_tiling_on_sc=True,
                                             needs_layout_passes=False),
        scratch_types=[
            pltpu.VMEM((n_chunks, _CHUNK), jnp.int32),
            pltpu.VMEM((n_chunks, _CHUNK), jnp.int32),
            pltpu.VMEM((n_chunks, _CHUNK), jnp.int32),
            pltpu.VMEM((n_chunks, _CHUNK), jnp.int32),
            pltpu.VMEM((2, _CHUNK, _LANE), jnp.float32),
            pltpu.VMEM((2, _CHUNK, _LANE), jnp.float32),
            pltpu.VMEM((b_per_w,), jnp.float32),
            pltpu.SemaphoreType.DMA,
            pltpu.SemaphoreType.DMA,
        ],
    )
    def k(uidx_hbm, iidx_hbm, utab_hbm, itab_hbm, out_hbm,
          uidx_v, iidx_v, urid_v, irid_v, ubuf, ibuf, out_v, sem0, sem1):
        sems = (sem0, sem1)
        wid = lax.axis_index("s") * _NC + lax.axis_index("c")
        base = wid * b_per_w

        for j in range(n_chunks):
            pltpu.sync_copy(uidx_hbm.at[pl.ds(base + j * _CHUNK, _CHUNK)],
                            uidx_v.at[j])
            pltpu.sync_copy(iidx_hbm.at[pl.ds(base + j * _CHUNK, _CHUNK)],
                            iidx_v.at[j])
            for g in range(groups):
                sl = pl.ds(g * _L, _L)
                urid_v[j, sl] = lax.shift_right_logical(uidx_v[j, sl], shift)
                irid_v[j, sl] = lax.shift_right_logical(iidx_v[j, sl], shift)

        def issue(j):
            p = j % 2
            cu = pltpu.async_copy(utab_hbm.at[urid_v.at[j]],
                                  ubuf.at[p], sems[p])
            ci = pltpu.async_copy(itab_hbm.at[irid_v.at[j]],
                                  ibuf.at[p], sems[p])
            return cu, ci

        lanes = lax.iota(jnp.int32, _L)
        cols = [jnp.full((_L,), d, jnp.int32) for d in range(D)]
        sub_mask = jnp.full((_L,), pack - 1, jnp.int32)
        dval = jnp.full((_L,), D, jnp.int32)

        pending = issue(0)
        for j in range(n_chunks):
            nxt = issue(j + 1) if j + 1 < n_chunks else None
            pending[0].wait()
            pending[1].wait()
            pending = nxt
            pbuf = jnp.full((_L,), j % 2, jnp.int32)

            def body(g, carry, j=j, pbuf=pbuf):
                sl = pl.ds(g * _L, _L)
                row = jnp.full((_L,), g * _L, jnp.int32) + lanes
                ucol = (uidx_v[j, sl] & sub_mask) * dval
                icol = (iidx_v[j, sl] & sub_mask) * dval
                accs = [jnp.zeros((_L,), jnp.float32) for _ in range(4)]
                for d in range(D):
                    u = plsc.load_gather(ubuf, [pbuf, row, ucol + cols[d]])
                    v = plsc.load_gather(ibuf, [pbuf, row, icol + cols[d]])
                    accs[d % 4] = accs[d % 4] + u * v
                out_v[pl.ds(j * _CHUNK + g * _L, _L)] = (
                    (accs[0] + accs[1]) + (accs[2] + accs[3]))
                return carry

            lax.fori_loop(0, groups, body, 0)

        pltpu.sync_copy(out_v, out_hbm.at[pl.ds(base, b_per_w)])

    return k


def kernel(user_item_tuple, user_table, item_table):
    uid = user_item_tuple[:, 0].astype(jnp.int32)
    iid = user_item_tuple[:, 1].astype(jnp.int32)
    n, d = user_table.shape
    pack = _LANE // d
    ut = user_table.reshape(n // pack, _LANE)
    it = item_table.reshape(n // pack, _LANE)
    return _build(uid.shape[0], d)(uid, iid, ut, it)
